# baseline (speedup 1.0000x reference)
import functools

import jax
import jax.numpy as jnp
from jax import lax
from jax.experimental import pallas as pl
from jax.experimental.pallas import tpu as pltpu
from jax.experimental.pallas import tpu_sc as plsc

N_ROWS = 4096
N_COLS = 200
B = N_ROWS * N_COLS
D = 128
V = 45
NC = 2
NS = 16
NW = NC * NS
BPW = B // NW
C = 128
NCHUNK = BPW // C
L = 16
NB = 4
A = 2
G = 160
NJ = G // NB
NBUILT = NCHUNK - G


def _embed_body(table_hbm, idxg_hbm, idxb_hbm, out_hbm,
                table_s, table_v, idx_v, idx_b, bufs, bbuf, sg, ss, sb):
    wid = lax.axis_index("s") * NC + lax.axis_index("c")
    base = wid * BPW

    @pl.when(lax.axis_index("s") == 0)
    def _():
        pltpu.sync_copy(table_hbm, table_s)

    pltpu.sync_copy(table_hbm, table_v)
    pltpu.sync_copy(idxg_hbm.at[wid], idx_v)
    pltpu.sync_copy(idxb_hbm.at[wid], idx_b)
    plsc.subcore_barrier()

    def gather(i, k):
        pltpu.async_copy(table_s.at[idx_v.at[i]], bufs[k], sg[k])

    def store(i, k):
        pltpu.async_copy(bufs[k], out_hbm.at[pl.ds(base + i * C, C)], ss[k])

    def wait_gather(i, k):
        pltpu.make_async_copy(table_s.at[idx_v.at[i]], bufs[k], sg[k]).wait()

    def wait_store(i, k):
        pltpu.make_async_copy(
            bufs[k], out_hbm.at[pl.ds(base + i * C, C)], ss[k]).wait()

    def build_quarter(j, q):
        @plsc.parallel_loop(q * (C // L // NB), (q + 1) * (C // L // NB))
        def _grp(g):
            idx16 = idx_b[pl.ds(j * C + g * L, L)]
            for l in range(L):
                k = idx16[l]
                r = g * L + l
                for c in range(D // L):
                    bbuf[r, pl.ds(c * L, L)] = table_v[k, pl.ds(c * L, L)]

    def store_built(j):
        pltpu.async_copy(
            bbuf, out_hbm.at[pl.ds(base + (G + j) * C, C)], sb)

    def wait_store_built(j):
        pltpu.make_async_copy(
            bbuf, out_hbm.at[pl.ds(base + (G + j) * C, C)], sb).wait()

    for i in range(A):
        gather(i, i)

    def body(j, carry):
        @pl.when(j > 0)
        def _():
            wait_store_built(j - 1)

        for k in range(NB):
            i = NB * j + k
            wait_gather(i, k)
            store(i, k)
            kA = (k + A) % NB
            if k < NB - A:
                @pl.when(j > 0)
                def _():
                    wait_store(i + A - NB, kA)

                gather(i + A, kA)
            else:
                @pl.when(j < NJ - 1)
                def _():
                    wait_store(i + A - NB, kA)
                    gather(i + A, kA)
            build_quarter(j, k)
        store_built(j)
        return carry

    lax.fori_loop(0, NJ, body, 0)
    wait_store_built(NBUILT - 1)
    for k in range(NB):
        wait_store(G - NB + k, k)


def kernel(inputs, table):
    flat = inputs.reshape(NW, BPW).astype(jnp.int32)
    idx_g = flat[:, : G * C].reshape(NW, G, C)
    idx_b = flat[:, G * C:].reshape(NW, NBUILT * C)
    mesh = plsc.VectorSubcoreMesh(core_axis_name="c", subcore_axis_name="s")
    out = pl.kernel(
        _embed_body,
        mesh=mesh,
        out_type=jax.ShapeDtypeStruct((B, D), jnp.float32),
        compiler_params=pltpu.CompilerParams(needs_layout_passes=False),
        scratch_types=[
            pltpu.VMEM_SHARED((V, D), jnp.float32),
            pltpu.VMEM((V, D), jnp.float32),
            pltpu.VMEM((G, C), jnp.int32),
            pltpu.VMEM((NBUILT * C,), jnp.int32),
            [pltpu.VMEM((C, D), jnp.float32) for _ in range(NB)],
            pltpu.VMEM((C, D), jnp.float32),
            [pltpu.SemaphoreType.DMA for _ in range(NB)],
            [pltpu.SemaphoreType.DMA for _ in range(NB)],
            pltpu.SemaphoreType.DMA,
        ],
    )(table, idx_g, idx_b)
    return out.reshape(N_ROWS, N_COLS, D)


# speedup vs baseline: 1.0404x; 1.0404x over previous
import functools

import jax
import jax.numpy as jnp
from jax import lax
from jax.experimental import pallas as pl
from jax.experimental.pallas import tpu as pltpu
from jax.experimental.pallas import tpu_sc as plsc

N_ROWS = 4096
N_COLS = 200
B = N_ROWS * N_COLS
D = 128
V = 45
NC = 2
NS = 16
NW = NC * NS
BPW = B // NW
C = 128
NCHUNK = BPW // C
L = 16
NB = 4
A = 3
G = 160
NJ = G // NB
NBUILT = NCHUNK - G


def _embed_body(table_hbm, idxg_hbm, idxb_hbm, out_hbm,
                table_s, table_v, idx_v, idx_b, bufs, bbuf, sg, ss, sb):
    wid = lax.axis_index("s") * NC + lax.axis_index("c")
    base = wid * BPW

    @pl.when(lax.axis_index("s") == 0)
    def _():
        pltpu.sync_copy(table_hbm, table_s)

    pltpu.sync_copy(table_hbm, table_v)
    pltpu.sync_copy(idxg_hbm.at[wid], idx_v)
    pltpu.sync_copy(idxb_hbm.at[wid], idx_b)
    plsc.subcore_barrier()

    def gather(i, k):
        pltpu.async_copy(table_s.at[idx_v.at[i]], bufs[k], sg[k])

    def store(i, k):
        pltpu.async_copy(bufs[k], out_hbm.at[pl.ds(base + i * C, C)], ss[k])

    def wait_gather(i, k):
        pltpu.make_async_copy(table_s.at[idx_v.at[i]], bufs[k], sg[k]).wait()

    def wait_store(i, k):
        pltpu.make_async_copy(
            bufs[k], out_hbm.at[pl.ds(base + i * C, C)], ss[k]).wait()

    def build_chunk(j):
        @plsc.parallel_loop(0, C // L)
        def _grp(g):
            idx16 = idx_b[pl.ds(j * C + g * L, L)]
            for l in range(L):
                k = idx16[l]
                r = g * L + l
                for c in range(D // L):
                    bbuf[r, pl.ds(c * L, L)] = table_v[k, pl.ds(c * L, L)]

    def store_built(j):
        pltpu.async_copy(
            bbuf, out_hbm.at[pl.ds(base + (G + j) * C, C)], sb)

    def wait_store_built(j):
        pltpu.make_async_copy(
            bbuf, out_hbm.at[pl.ds(base + (G + j) * C, C)], sb).wait()

    for i in range(A):
        gather(i, i)

    def body(j, carry):
        @pl.when(j > 0)
        def _():
            wait_store_built(j - 1)

        build_chunk(j)
        store_built(j)

        for k in range(NB):
            i = NB * j + k
            wait_gather(i, k)
            store(i, k)
            kA = (k + A) % NB
            if k < NB - A:
                @pl.when(j > 0)
                def _():
                    wait_store(i + A - NB, kA)

                gather(i + A, kA)
            else:
                @pl.when(j < NJ - 1)
                def _():
                    wait_store(i + A - NB, kA)
                    gather(i + A, kA)
        return carry

    lax.fori_loop(0, NJ, body, 0)
    wait_store_built(NBUILT - 1)
    for k in range(NB):
        wait_store(G - NB + k, k)


def kernel(inputs, table):
    flat = inputs.reshape(NW, BPW).astype(jnp.int32)
    idx_g = flat[:, : G * C].reshape(NW, G, C)
    idx_b = flat[:, G * C:].reshape(NW, NBUILT * C)
    mesh = plsc.VectorSubcoreMesh(core_axis_name="c", subcore_axis_name="s")
    out = pl.kernel(
        _embed_body,
        mesh=mesh,
        out_type=jax.ShapeDtypeStruct((B, D), jnp.float32),
        compiler_params=pltpu.CompilerParams(needs_layout_passes=False),
        scratch_types=[
            pltpu.VMEM_SHARED((V, D), jnp.float32),
            pltpu.VMEM((V, D), jnp.float32),
            pltpu.VMEM((G, C), jnp.int32),
            pltpu.VMEM((NBUILT * C,), jnp.int32),
            [pltpu.VMEM((C, D), jnp.float32) for _ in range(NB)],
            pltpu.VMEM((C, D), jnp.float32),
            [pltpu.SemaphoreType.DMA for _ in range(NB)],
            [pltpu.SemaphoreType.DMA for _ in range(NB)],
            pltpu.SemaphoreType.DMA,
        ],
    )(table, idx_g, idx_b)
    return out.reshape(N_ROWS, N_COLS, D)


# speedup vs baseline: 1.1391x; 1.0949x over previous
import functools

import jax
import jax.numpy as jnp
from jax import lax
from jax.experimental import pallas as pl
from jax.experimental.pallas import tpu as pltpu
from jax.experimental.pallas import tpu_sc as plsc

N_ROWS = 4096
N_COLS = 200
B = N_ROWS * N_COLS
D = 128
V = 45
NC = 2
NS = 16
NW = NC * NS
BPW = B // NW
C = 128
NCHUNK = BPW // C
NB = 5
A = 3
NJ = NCHUNK // NB


def _embed_body(table_hbm, idx_hbm, out_hbm, table_s, idx_v, bufs, sg, ss):
    wid = lax.axis_index("s") * NC + lax.axis_index("c")
    base = wid * BPW

    @pl.when(lax.axis_index("s") == 0)
    def _():
        pltpu.sync_copy(table_hbm, table_s)

    pltpu.sync_copy(idx_hbm.at[wid], idx_v)
    plsc.subcore_barrier()

    def gather(i, k):
        pltpu.async_copy(table_s.at[idx_v.at[i]], bufs[k], sg[k])

    def store(i, k):
        pltpu.async_copy(bufs[k], out_hbm.at[pl.ds(base + i * C, C)], ss[k])

    def wait_gather(i, k):
        pltpu.make_async_copy(table_s.at[idx_v.at[i]], bufs[k], sg[k]).wait()

    def wait_store(i, k):
        pltpu.make_async_copy(
            bufs[k], out_hbm.at[pl.ds(base + i * C, C)], ss[k]).wait()

    for i in range(A):
        gather(i, i)

    def body(j, carry):
        for k in range(NB):
            i = NB * j + k
            wait_gather(i, k)
            store(i, k)
            kA = (k + A) % NB
            if k < NB - A:
                @pl.when(j > 0)
                def _():
                    wait_store(i + A - NB, kA)

                gather(i + A, kA)
            else:
                @pl.when(j < NJ - 1)
                def _():
                    wait_store(i + A - NB, kA)
                    gather(i + A, kA)
        return carry

    lax.fori_loop(0, NJ, body, 0)
    for k in range(NB):
        wait_store(NCHUNK - NB + k, k)


def kernel(inputs, table):
    idx = inputs.reshape(NW, NCHUNK, C).astype(jnp.int32)
    mesh = plsc.VectorSubcoreMesh(core_axis_name="c", subcore_axis_name="s")
    out = pl.kernel(
        _embed_body,
        mesh=mesh,
        out_type=jax.ShapeDtypeStruct((B, D), jnp.float32),
        compiler_params=pltpu.CompilerParams(needs_layout_passes=False),
        scratch_types=[
            pltpu.VMEM_SHARED((V, D), jnp.float32),
            pltpu.VMEM((NCHUNK, C), jnp.int32),
            [pltpu.VMEM((C, D), jnp.float32) for _ in range(NB)],
            [pltpu.SemaphoreType.DMA for _ in range(NB)],
            [pltpu.SemaphoreType.DMA for _ in range(NB)],
        ],
    )(table, idx)
    return out.reshape(N_ROWS, N_COLS, D)
